# baseline (speedup 1.0000x reference)
import functools

import jax
import jax.numpy as jnp
from jax import lax
from jax.experimental import pallas as pl
from jax.experimental.pallas import tpu as pltpu
from jax.experimental.pallas import tpu_sc as plsc

VAR_LEN = 1000000
EMBED = 64
B = 4096
S = 200

NC = 2
NS = 16
NW = NC * NS

N = B * S
N_PER_W = N // NW
CH = 128
N_CHUNKS = N_PER_W // CH
LANES = 16


def _body(x_hbm, idx_hbm, pos_hbm, table_hbm, out_hbm,
          idx_v, rows_v, x_v, pos_v, gsem, xsem, psem):
    wid = lax.axis_index("s") * NC + lax.axis_index("c")
    base = wid * N_PER_W

    def chunk(j, _):
        row0 = base + j * CH
        off = row0 * EMBED
        pltpu.sync_copy(idx_hbm.at[pl.ds(row0, CH)], idx_v)
        g = pltpu.async_copy(table_hbm.at[idx_v], rows_v, gsem)
        cx = pltpu.async_copy(x_hbm.at[pl.ds(off, CH * EMBED)], x_v, xsem)
        cp = pltpu.async_copy(pos_hbm.at[pl.ds(off, CH * EMBED)], pos_v, psem)
        g.wait()
        cx.wait()
        cp.wait()

        def add_row(r, _):
            for k in range(EMBED // LANES):
                s = pl.ds(r * EMBED + k * LANES, LANES)
                x_v[s] = x_v[s] + pos_v[s] + rows_v[r, pl.ds(k * LANES, LANES)]
            return ()

        lax.fori_loop(0, CH, add_row, (), unroll=2)
        pltpu.sync_copy(x_v, out_hbm.at[pl.ds(off, CH * EMBED)])
        return ()

    lax.fori_loop(0, N_CHUNKS, chunk, ())


@jax.jit
def _run(xf, idx, pf, table):
    mesh = plsc.VectorSubcoreMesh(
        core_axis_name="c", subcore_axis_name="s",
        num_cores=NC, num_subcores=NS)
    fn = pl.kernel(
        _body,
        out_type=jax.ShapeDtypeStruct((N * EMBED,), jnp.float32),
        mesh=mesh,
        scratch_types=[
            pltpu.VMEM((CH,), jnp.int32),
            pltpu.VMEM((CH, EMBED), jnp.float32),
            pltpu.VMEM((CH * EMBED,), jnp.float32),
            pltpu.VMEM((CH * EMBED,), jnp.float32),
            pltpu.SemaphoreType.DMA,
            pltpu.SemaphoreType.DMA,
            pltpu.SemaphoreType.DMA,
        ],
        compiler_params=pltpu.CompilerParams(use_tc_tiling_on_sc=False),
    )
    return fn(xf, idx, pf, table)


def kernel(x, variable_seq, pos_emb, table):
    xf = x.reshape(-1)
    pf = pos_emb.reshape(-1)
    idx = variable_seq.reshape(-1).astype(jnp.int32)
    out = _run(xf, idx, pf, table)
    return out.reshape(B, S, EMBED)


# speedup vs baseline: 1.1625x; 1.1625x over previous
import jax
import jax.numpy as jnp
from jax import lax
from jax.experimental import pallas as pl
from jax.experimental.pallas import tpu as pltpu
from jax.experimental.pallas import tpu_sc as plsc

VAR_LEN = 1000000
EMBED = 64
B = 4096
S = 200

NC = 2
NS = 16
NW = NC * NS

N = B * S
N_PER_W = N // NW
CH = 128
N_CHUNKS = N_PER_W // CH
LANES = 16
CB = CH * EMBED


def _body(x_hbm, idx_hbm, pos_hbm, table_hbm, out_hbm,
          idx_v,
          rows_a, x_a, pos_a, out_a,
          rows_b, x_b, pos_b, out_b,
          in_sem_a, in_sem_b, out_sem_a, out_sem_b):
    wid = lax.axis_index("s") * NC + lax.axis_index("c")
    base = wid * N_PER_W

    bufs = (
        (rows_a, x_a, pos_a, out_a, in_sem_a, out_sem_a),
        (rows_b, x_b, pos_b, out_b, in_sem_b, out_sem_b),
    )

    def issue_in(j, rows, xb, pb, sem):
        off = (base + j * CH) * EMBED
        pltpu.async_copy(table_hbm.at[idx_v.at[j]], rows, sem)
        pltpu.async_copy(x_hbm.at[pl.ds(off, CB)], xb, sem)
        pltpu.async_copy(pos_hbm.at[pl.ds(off, CB)], pb, sem)

    def wait_in(j, rows, xb, pb, sem):
        pltpu.make_async_copy(table_hbm.at[idx_v.at[j]], rows, sem).wait()
        pltpu.make_async_copy(x_hbm.at[pl.ds(0, CB)], xb, sem).wait()
        pltpu.make_async_copy(pos_hbm.at[pl.ds(0, CB)], pb, sem).wait()

    def compute(rows, xb, pb, ob):
        def add_row(r, _):
            for k in range(EMBED // LANES):
                s = pl.ds(r * EMBED + k * LANES, LANES)
                ob[s] = xb[s] + pb[s] + rows[r, pl.ds(k * LANES, LANES)]
            return ()
        lax.fori_loop(0, CH, add_row, (), unroll=2)

    pltpu.sync_copy(idx_hbm.at[wid], idx_v)

    issue_in(0, rows_a, x_a, pos_a, in_sem_a)
    issue_in(1, rows_b, x_b, pos_b, in_sem_b)

    def iter_body(t, _):
        for half in range(2):
            rows, xb, pb, ob, isem, osem = bufs[half]
            j = 2 * t + half
            off = (base + j * CH) * EMBED
            wait_in(j, rows, xb, pb, isem)

            @pl.when(t > 0)
            def _():
                pltpu.make_async_copy(
                    ob, out_hbm.at[pl.ds(0, CB)], osem).wait()

            compute(rows, xb, pb, ob)
            pltpu.async_copy(ob, out_hbm.at[pl.ds(off, CB)], osem)

            @pl.when(j + 2 < N_CHUNKS)
            def _():
                issue_in(j + 2, rows, xb, pb, isem)
        return ()

    lax.fori_loop(0, N_CHUNKS // 2, iter_body, ())

    pltpu.make_async_copy(out_a, out_hbm.at[pl.ds(0, CB)], out_sem_a).wait()
    pltpu.make_async_copy(out_b, out_hbm.at[pl.ds(0, CB)], out_sem_b).wait()


@jax.jit
def _run(xf, idx3, pf, table):
    mesh = plsc.VectorSubcoreMesh(
        core_axis_name="c", subcore_axis_name="s",
        num_cores=NC, num_subcores=NS)
    fn = pl.kernel(
        _body,
        out_type=jax.ShapeDtypeStruct((N * EMBED,), jnp.float32),
        mesh=mesh,
        scratch_types=[
            pltpu.VMEM((N_CHUNKS, CH), jnp.int32),
            pltpu.VMEM((CH, EMBED), jnp.float32),
            pltpu.VMEM((CB,), jnp.float32),
            pltpu.VMEM((CB,), jnp.float32),
            pltpu.VMEM((CB,), jnp.float32),
            pltpu.VMEM((CH, EMBED), jnp.float32),
            pltpu.VMEM((CB,), jnp.float32),
            pltpu.VMEM((CB,), jnp.float32),
            pltpu.VMEM((CB,), jnp.float32),
            pltpu.SemaphoreType.DMA,
            pltpu.SemaphoreType.DMA,
            pltpu.SemaphoreType.DMA,
            pltpu.SemaphoreType.DMA,
        ],
        compiler_params=pltpu.CompilerParams(use_tc_tiling_on_sc=False),
    )
    return fn(xf, idx3, pf, table)


def kernel(x, variable_seq, pos_emb, table):
    xf = x.reshape(-1)
    pf = pos_emb.reshape(-1)
    idx3 = variable_seq.reshape(NW, N_CHUNKS, CH).astype(jnp.int32)
    out = _run(xf, idx3, pf, table)
    return out.reshape(B, S, EMBED)


# speedup vs baseline: 2.7057x; 2.3275x over previous
import jax
import jax.numpy as jnp
from jax import lax
from jax.experimental import pallas as pl
from jax.experimental.pallas import tpu as pltpu
from jax.experimental.pallas import tpu_sc as plsc

VAR_LEN = 1000000
EMBED = 64
B = 4096
S = 200

NC = 2
NS = 16
NW = NC * NS

LANES = 16
ET = EMBED // 8
BT = B // 128
UW = 8 * 128


def _sc_body(x_hbm, idx_hbm, pos_hbm, table_hbm, out_hbm,
             idx_v, rows_a, o_a, p_a, rows_b, o_b, p_b,
             in_sem_a, in_sem_b, out_sem_a, out_sem_b):
    w = lax.axis_index("s") * NC + lax.axis_index("c")

    bufs = (
        (rows_a, o_a, p_a, in_sem_a, out_sem_a),
        (rows_b, o_b, p_b, in_sem_b, out_sem_b),
    )

    pltpu.sync_copy(idx_hbm.at[:, w], idx_v)

    def issue_in(s, rows, ou, pu, sem):
        pltpu.async_copy(table_hbm.at[idx_v.at[s // 8, s % 8]], rows, sem)
        pltpu.async_copy(x_hbm.at[pl.ds(s * ET, ET), w], ou, sem)
        pltpu.async_copy(pos_hbm.at[pl.ds(s * ET, ET), w], pu, sem)

    def wait_in(rows, ou, pu, sem):
        pltpu.make_async_copy(table_hbm.at[idx_v.at[0, 0]], rows, sem).wait()
        pltpu.make_async_copy(x_hbm.at[pl.ds(0, ET), 0], ou, sem).wait()
        pltpu.make_async_copy(pos_hbm.at[pl.ds(0, ET), 0], pu, sem).wait()

    def compute(rows, ou, pu):
        for et in range(ET):
            for k8 in range(UW // (8 * LANES)):
                sls = [pl.ds((8 * k8 + j) * LANES, LANES) for j in range(8)]
                ps = [pu[et, sl] for sl in sls]
                for j in range(8):
                    plsc.addupdate(ou.at[et, sls[j]], ps[j])

        lane = lax.iota(jnp.int32, LANES)
        rvs = [lane + (LANES * k) for k in range(8)]

        def step(c0, ev):
            eh = ev >> 3
            wl0 = ((ev & 7) << 7) + lane
            gs = [plsc.load_gather(rows, [rvs[k], ev]) for k in range(8)]
            for k in range(8):
                plsc.addupdate_scatter(ou, [eh, wl0 + LANES * k], gs[k])
            return (ev + 1) & (EMBED - 1)

        lax.fori_loop(0, EMBED, step, lane)

    issue_in(0, rows_a, o_a, p_a, in_sem_a)
    issue_in(1, rows_b, o_b, p_b, in_sem_b)

    def iter_body(t, _):
        for half in range(2):
            rows, ou, pu, isem, osem = bufs[half]
            s = 2 * t + half
            wait_in(rows, ou, pu, isem)

            @pl.when(t > 0)
            def _():
                pltpu.make_async_copy(
                    ou, out_hbm.at[pl.ds(0, ET), 0], osem).wait()

            compute(rows, ou, pu)
            pltpu.async_copy(ou, out_hbm.at[pl.ds(s * ET, ET), w], osem)

            @pl.when(s + 2 < S)
            def _():
                issue_in(s + 2, rows, ou, pu, isem)
        return ()

    lax.fori_loop(0, S // 2, iter_body, ())

    pltpu.make_async_copy(o_a, out_hbm.at[pl.ds(0, ET), 0], out_sem_a).wait()
    pltpu.make_async_copy(o_b, out_hbm.at[pl.ds(0, ET), 0], out_sem_b).wait()


@jax.jit
def _run(x, variable_seq, pos_emb, table):
    def phys3(a):
        return (jnp.transpose(a, (1, 2, 0))
                .reshape(S, ET, 8, BT, 128)
                .transpose(0, 1, 3, 2, 4)
                .reshape(S * ET, BT, UW))

    x3 = phys3(x)
    p3 = phys3(pos_emb)

    idx4 = (jnp.transpose(variable_seq.astype(jnp.int32), (1, 0))
            .reshape(S // 8, 8, BT, 128)
            .transpose(0, 2, 1, 3))

    mesh = plsc.VectorSubcoreMesh(
        core_axis_name="c", subcore_axis_name="s",
        num_cores=NC, num_subcores=NS)
    out3 = pl.kernel(
        _sc_body,
        out_type=jax.ShapeDtypeStruct((S * ET, BT, UW), jnp.float32),
        mesh=mesh,
        scratch_types=[
            pltpu.VMEM((S // 8, 8, 128), jnp.int32),
            pltpu.VMEM((128, EMBED), jnp.float32),
            pltpu.VMEM((ET, UW), jnp.float32),
            pltpu.VMEM((ET, UW), jnp.float32),
            pltpu.VMEM((128, EMBED), jnp.float32),
            pltpu.VMEM((ET, UW), jnp.float32),
            pltpu.VMEM((ET, UW), jnp.float32),
            pltpu.SemaphoreType.DMA,
            pltpu.SemaphoreType.DMA,
            pltpu.SemaphoreType.DMA,
            pltpu.SemaphoreType.DMA,
        ],
        compiler_params=pltpu.CompilerParams(
            use_tc_tiling_on_sc=False, needs_layout_passes=False),
    )(x3, idx4, p3, table)

    out = (out3.reshape(S, ET, BT, 8, 128)
           .transpose(0, 1, 3, 2, 4)
           .reshape(S, EMBED, B)
           .transpose(2, 0, 1))
    return out


def kernel(x, variable_seq, pos_emb, table):
    return _run(x, variable_seq, pos_emb, table)


# speedup vs baseline: 2.9009x; 1.0722x over previous
import jax
import jax.numpy as jnp
from jax import lax
from jax.experimental import pallas as pl
from jax.experimental.pallas import tpu as pltpu
from jax.experimental.pallas import tpu_sc as plsc

VAR_LEN = 1000000
EMBED = 64
B = 4096
S = 200

NC = 2
NS = 16
NW = NC * NS

LANES = 16
ET = EMBED // 8
BT = B // 128
UW = 8 * 128


def _tc_add_body(x_ref, p_ref, o_ref):
    o_ref[...] = x_ref[...] + p_ref[...]


def _sc_body(base_hbm, idx_hbm, table_hbm, out_hbm,
             idx_v, rows_a, unit_a, rows_b, unit_b,
             in_sem_a, in_sem_b, out_sem_a, out_sem_b):
    w = lax.axis_index("s") * NC + lax.axis_index("c")

    bufs = (
        (rows_a, unit_a, in_sem_a, out_sem_a),
        (rows_b, unit_b, in_sem_b, out_sem_b),
    )

    pltpu.sync_copy(idx_hbm.at[:, w], idx_v)

    def issue_in(s, rows, unit, sem):
        pltpu.async_copy(table_hbm.at[idx_v.at[s // 8, s % 8]], rows, sem)
        pltpu.async_copy(base_hbm.at[pl.ds(s * ET, ET), w], unit, sem)

    def wait_in(rows, unit, sem):
        pltpu.make_async_copy(table_hbm.at[idx_v.at[0, 0]], rows, sem).wait()
        pltpu.make_async_copy(base_hbm.at[pl.ds(0, ET), 0], unit, sem).wait()

    def compute(rows, unit):
        lane = lax.iota(jnp.int32, LANES)
        rvs = [lane + (LANES * k) for k in range(8)]

        def step(c0, ev):
            eh = ev >> 3
            wl0 = ((ev & 7) << 7) + lane
            gs = [plsc.load_gather(rows, [rvs[k], ev]) for k in range(8)]
            for k in range(8):
                plsc.addupdate_scatter(unit, [eh, wl0 + LANES * k], gs[k])
            return (ev + 1) & (EMBED - 1)

        lax.fori_loop(0, EMBED, step, lane)

    issue_in(0, rows_a, unit_a, in_sem_a)
    issue_in(1, rows_b, unit_b, in_sem_b)

    def iter_body(t, _):
        for half in range(2):
            rows, unit, isem, osem = bufs[half]
            s = 2 * t + half
            wait_in(rows, unit, isem)

            @pl.when(t > 0)
            def _():
                pltpu.make_async_copy(
                    unit, out_hbm.at[pl.ds(0, ET), 0], osem).wait()

            compute(rows, unit)
            pltpu.async_copy(unit, out_hbm.at[pl.ds(s * ET, ET), w], osem)

            @pl.when(s + 2 < S)
            def _():
                issue_in(s + 2, rows, unit, isem)
        return ()

    lax.fori_loop(0, S // 2, iter_body, ())

    pltpu.make_async_copy(unit_a, out_hbm.at[pl.ds(0, ET), 0], out_sem_a).wait()
    pltpu.make_async_copy(unit_b, out_hbm.at[pl.ds(0, ET), 0], out_sem_b).wait()


@jax.jit
def _run(x, variable_seq, pos_emb, table):
    def phys5(a):
        return (jnp.transpose(a, (1, 2, 0))
                .reshape(S, ET, 8, BT, 128)
                .transpose(0, 1, 3, 2, 4))

    x2 = phys5(x).reshape(S * ET * BT * 8, 128)
    p2 = phys5(pos_emb).reshape(x2.shape)

    idx4 = (jnp.transpose(variable_seq.astype(jnp.int32), (1, 0))
            .reshape(S // 8, 8, BT, 128)
            .transpose(0, 2, 1, 3))


    grid = 100
    rows_per = x2.shape[0] // grid
    base2 = pl.pallas_call(
        _tc_add_body,
        grid=(grid,),
        in_specs=[
            pl.BlockSpec((rows_per, 128), lambda i: (i, 0)),
            pl.BlockSpec((rows_per, 128), lambda i: (i, 0)),
        ],
        out_specs=pl.BlockSpec((rows_per, 128), lambda i: (i, 0)),
        out_shape=jax.ShapeDtypeStruct(x2.shape, jnp.float32),
    )(x2, p2)

    base3 = base2.reshape(S * ET, BT, UW)

    mesh = plsc.VectorSubcoreMesh(
        core_axis_name="c", subcore_axis_name="s",
        num_cores=NC, num_subcores=NS)
    out3 = pl.kernel(
        _sc_body,
        out_type=jax.ShapeDtypeStruct((S * ET, BT, UW), jnp.float32),
        mesh=mesh,
        scratch_types=[
            pltpu.VMEM((S // 8, 8, 128), jnp.int32),
            pltpu.VMEM((128, EMBED), jnp.float32),
            pltpu.VMEM((ET, UW), jnp.float32),
            pltpu.VMEM((128, EMBED), jnp.float32),
            pltpu.VMEM((ET, UW), jnp.float32),
            pltpu.SemaphoreType.DMA,
            pltpu.SemaphoreType.DMA,
            pltpu.SemaphoreType.DMA,
            pltpu.SemaphoreType.DMA,
        ],
        compiler_params=pltpu.CompilerParams(
            use_tc_tiling_on_sc=False, needs_layout_passes=False),
    )(base3, idx4, table)

    out = (out3.reshape(S, ET, BT, 8, 128)
           .transpose(0, 1, 3, 2, 4)
           .reshape(S, EMBED, B)
           .transpose(2, 0, 1))
    return out


def kernel(x, variable_seq, pos_emb, table):
    return _run(x, variable_seq, pos_emb, table)
